# baseline (speedup 1.0000x reference)
import functools

import jax
import jax.numpy as jnp
from jax.experimental import pallas as pl
from jax.experimental.pallas import tpu as pltpu
from jax.experimental.pallas import tpu_sc as plsc

_B = 512
_TN = 1024


def _routing_metadata(topk_ids, topk_weights, e_num, block_rows):
    m, topk = topk_ids.shape
    s = m * topk
    flat_e = topk_ids.reshape(s).astype(jnp.int32)
    chunk = 128
    g = s // chunk
    ohf = (flat_e[:, None] == jnp.arange(e_num, dtype=jnp.int32)[None, :]
           ).astype(jnp.float32)
    ohc = ohf.reshape(g, chunk, e_num)
    tri = jnp.tril(jnp.ones((chunk, chunk), jnp.float32))
    inner = jax.lax.dot_general(tri, ohc, (((1,), (1,)), ((), ())))
    inner = inner.transpose(1, 0, 2)
    chunk_tot = ohc.sum(axis=1)
    outer = jnp.cumsum(chunk_tot, axis=0) - chunk_tot
    prefix_f = (inner + outer[:, None, :]).reshape(s, e_num)
    counts = prefix_f[-1].astype(jnp.int32)
    blocks_e = (counts + block_rows - 1) // block_rows
    block_bound = jnp.cumsum(blocks_e)
    pad_start = (block_bound - blocks_e) * block_rows
    rank_f = jnp.sum(prefix_f * ohf, axis=1) - 1.0
    pos = (jnp.sum(pad_start.astype(jnp.float32)[None, :] * ohf, axis=1)
           + rank_f).astype(jnp.int32)
    p_total = s + e_num * block_rows
    nb = p_total // block_rows
    block_expert = jnp.minimum(
        jnp.sum((block_bound[None, :] <=
                 jnp.arange(nb, dtype=jnp.int32)[:, None]).astype(jnp.int32),
                axis=1),
        e_num - 1).astype(jnp.int32)
    num_used = block_bound[-1].astype(jnp.int32)
    return pos, block_expert, num_used


def _sc_dispatch_rows(table, pos, p_total):
    m, k = table.shape
    topk = pos.shape[0] // m
    info = plsc.get_sparse_core_info()
    nw = info.num_cores * info.num_subcores
    toks_w = m // nw
    ct = 16
    chunks = toks_w // ct
    pos4 = pos.reshape(nw, chunks, ct, topk).transpose(0, 1, 3, 2)
    mesh = plsc.VectorSubcoreMesh(core_axis_name="c", subcore_axis_name="s")

    @functools.partial(
        pl.kernel, mesh=mesh,
        out_type=jax.ShapeDtypeStruct((p_total, k), table.dtype),
        scratch_types=[pltpu.VMEM((chunks, topk, ct), jnp.int32)]
                      + [pltpu.VMEM((ct, k), table.dtype) for _ in range(2)]
                      + [pltpu.SemaphoreType.DMA for _ in range(2 + 2 * topk)])
    def kern(table_hbm, idx_hbm, out_hbm, idx_v, *rest):
        bufs = rest[:2]
        lsems = rest[2:4]
        ssems = (rest[4:4 + topk], rest[4 + topk:4 + 2 * topk])
        wid = jax.lax.axis_index("s") * info.num_cores + jax.lax.axis_index("c")
        base = wid * toks_w
        pltpu.sync_copy(idx_hbm.at[wid], idx_v)
        lh = [None] * chunks
        sh = [None] * chunks
        for c in range(chunks):
            if c >= 2:
                for h in sh[c - 2]:
                    h.wait()
            lh[c] = pltpu.async_copy(
                table_hbm.at[pl.ds(base + c * ct, ct)], bufs[c % 2],
                lsems[c % 2])
            if c >= 1:
                lh[c - 1].wait()
                sh[c - 1] = [
                    pltpu.async_copy(bufs[(c - 1) % 2],
                                     out_hbm.at[idx_v.at[c - 1, t]],
                                     ssems[(c - 1) % 2][t])
                    for t in range(topk)]
        lh[chunks - 1].wait()
        sh[chunks - 1] = [
            pltpu.async_copy(bufs[(chunks - 1) % 2],
                             out_hbm.at[idx_v.at[chunks - 1, t]],
                             ssems[(chunks - 1) % 2][t])
            for t in range(topk)]
        for c in (chunks - 2, chunks - 1):
            for h in sh[c]:
                h.wait()

    return kern(table, pos4)


def _sc_combine_rows(yw, pos, topk_weights, m, topk):
    k = yw.shape[1]
    info = plsc.get_sparse_core_info()
    nw = info.num_cores * info.num_subcores
    nl = info.num_lanes
    toks_w = m // nw
    cc = 8
    chunks = toks_w // cc
    pos4 = pos.reshape(nw, chunks, cc, topk).transpose(0, 1, 3, 2)
    wbc = jnp.broadcast_to(topk_weights.reshape(m, topk, 1), (m, topk, nl))
    wbc = wbc.reshape(nw, chunks, cc, topk, nl).transpose(0, 1, 3, 2, 4)
    mesh = plsc.VectorSubcoreMesh(core_axis_name="c", subcore_axis_name="s")

    @functools.partial(
        pl.kernel, mesh=mesh,
        out_type=jax.ShapeDtypeStruct((m, k), yw.dtype),
        scratch_types=[pltpu.VMEM((chunks, topk, cc), jnp.int32),
                       pltpu.VMEM((chunks, topk, cc, nl), jnp.float32)]
                      + [pltpu.VMEM((cc, k), jnp.float32)
                         for _ in range(2 * topk)]
                      + [pltpu.SemaphoreType.DMA for _ in range(2 * topk + 2)])
    def kern(y_hbm, p_hbm, w_hbm, o_hbm, idx_v, w_v, *rest):
        bufs = (rest[:topk], rest[topk:2 * topk])
        gsems = (rest[2 * topk:3 * topk], rest[3 * topk:4 * topk])
        wsems = rest[4 * topk:4 * topk + 2]
        wid = jax.lax.axis_index("s") * info.num_cores + jax.lax.axis_index("c")
        base = wid * toks_w
        pltpu.sync_copy(p_hbm.at[wid], idx_v)
        pltpu.sync_copy(w_hbm.at[wid], w_v)
        gh = [None] * chunks
        wh = [None] * chunks

        def accumulate(c):
            bb = bufs[c % 2]

            @pl.loop(0, cc)
            def _(r):
                @pl.loop(0, k, step=nl)
                def _(col):
                    slc = (pl.ds(r, 1), pl.ds(col, nl))
                    acc = (bb[0].at[*slc][...]
                           * w_v.at[c, 0, pl.ds(r, 1)][...])
                    for t in range(1, topk):
                        acc = acc + (bb[t].at[*slc][...]
                                     * w_v.at[c, t, pl.ds(r, 1)][...])
                    bb[0].at[*slc][...] = acc

        for c in range(chunks):
            if c >= 2:
                wh[c - 2].wait()
            gh[c] = [pltpu.async_copy(y_hbm.at[idx_v.at[c, t]],
                                      bufs[c % 2][t], gsems[c % 2][t])
                     for t in range(topk)]
            if c >= 1:
                for h in gh[c - 1]:
                    h.wait()
                accumulate(c - 1)
                wh[c - 1] = pltpu.async_copy(
                    bufs[(c - 1) % 2][0],
                    o_hbm.at[pl.ds(base + (c - 1) * cc, cc)], wsems[(c - 1) % 2])
        for h in gh[chunks - 1]:
            h.wait()
        accumulate(chunks - 1)
        wh[chunks - 1] = pltpu.async_copy(
            bufs[(chunks - 1) % 2][0],
            o_hbm.at[pl.ds(base + (chunks - 1) * cc, cc)], wsems[(chunks - 1) % 2])
        wh[chunks - 2].wait()
        wh[chunks - 1].wait()

    return kern(yw, pos4, wbc)


def _expert_gemm(xw, w1, w2, block_expert, num_used):
    p_total, k = xw.shape
    e_num, n, _ = w1.shape
    d_ff = n // 2
    t_steps = d_ff // _TN
    nb = p_total // _B

    def body(be_ref, nu_ref, x_ref, w1g_ref, w1u_ref, w2_ref, o_ref,
             acc_ref):
        b = pl.program_id(0)
        t = pl.program_id(1)

        @pl.when(b < nu_ref[0])
        def _():
            x = x_ref[...]
            g = jax.lax.dot_general(x, w1g_ref[0], (((1,), (1,)), ((), ())),
                                    preferred_element_type=jnp.float32)
            u = jax.lax.dot_general(x, w1u_ref[0], (((1,), (1,)), ((), ())),
                                    preferred_element_type=jnp.float32)
            act = g * jax.nn.sigmoid(g) * u
            y = jax.lax.dot_general(act, w2_ref[0], (((1,), (1,)), ((), ())),
                                    preferred_element_type=jnp.float32)

            @pl.when(t == 0)
            def _():
                acc_ref[...] = y

            @pl.when(t != 0)
            def _():
                acc_ref[...] += y

            @pl.when(t == t_steps - 1)
            def _():
                o_ref[...] = acc_ref[...]

    def beff(b, nu):
        return jnp.minimum(b, nu[0] - 1)

    def eeff(b, be, nu):
        return be[jnp.minimum(b, nu[0] - 1)]

    def teff(b, t, nu):
        return jnp.where(b < nu[0], t, t_steps - 1)

    grid_spec = pltpu.PrefetchScalarGridSpec(
        num_scalar_prefetch=2,
        grid=(nb, t_steps),
        in_specs=[
            pl.BlockSpec((_B, k), lambda b, t, be, nu: (beff(b, nu), 0)),
            pl.BlockSpec((1, _TN, k),
                         lambda b, t, be, nu: (eeff(b, be, nu), teff(b, t, nu), 0)),
            pl.BlockSpec((1, _TN, k),
                         lambda b, t, be, nu: (eeff(b, be, nu),
                                               t_steps + teff(b, t, nu), 0)),
            pl.BlockSpec((1, k, _TN),
                         lambda b, t, be, nu: (eeff(b, be, nu), 0, teff(b, t, nu))),
        ],
        out_specs=pl.BlockSpec((_B, k), lambda b, t, be, nu: (b, 0)),
        scratch_shapes=[pltpu.VMEM((_B, k), jnp.float32)],
    )
    return pl.pallas_call(
        body,
        grid_spec=grid_spec,
        out_shape=jax.ShapeDtypeStruct((p_total, k), jnp.float32),
        compiler_params=pltpu.CompilerParams(
            dimension_semantics=("parallel", "arbitrary")),
    )(block_expert, num_used.reshape(1), xw, w1, w1, w2)


def kernel(hidden_states, w1, w2, topk_weights, topk_ids):
    m, _ = hidden_states.shape
    e_num = w1.shape[0]
    topk = topk_ids.shape[1]
    s = m * topk
    pos, block_expert, num_used = _routing_metadata(
        topk_ids, topk_weights, e_num, _B)
    p_total = s + e_num * _B
    xw = _sc_dispatch_rows(hidden_states, pos, p_total)
    yw = _expert_gemm(xw, w1, w2, block_expert, num_used)
    return _sc_combine_rows(yw, pos, topk_weights, m, topk)


# speedup vs baseline: 1.0059x; 1.0059x over previous
import functools

import jax
import jax.numpy as jnp
from jax.experimental import pallas as pl
from jax.experimental.pallas import tpu as pltpu
from jax.experimental.pallas import tpu_sc as plsc

_B = 512
_TN = 1024


def _routing_metadata(topk_ids, topk_weights, e_num, block_rows):
    m, topk = topk_ids.shape
    s = m * topk
    flat_e = topk_ids.reshape(s).astype(jnp.int32)
    chunk = 128
    g = s // chunk
    ohf = (flat_e[:, None] == jnp.arange(e_num, dtype=jnp.int32)[None, :]
           ).astype(jnp.float32)
    ohc = ohf.reshape(g, chunk, e_num)
    tri = jnp.tril(jnp.ones((chunk, chunk), jnp.float32))
    inner = jax.lax.dot_general(tri, ohc, (((1,), (1,)), ((), ())))
    inner = inner.transpose(1, 0, 2)
    chunk_tot = ohc.sum(axis=1)
    outer = jnp.cumsum(chunk_tot, axis=0) - chunk_tot
    prefix_f = (inner + outer[:, None, :]).reshape(s, e_num)
    counts = prefix_f[-1].astype(jnp.int32)
    blocks_e = (counts + block_rows - 1) // block_rows
    block_bound = jnp.cumsum(blocks_e)
    pad_start = (block_bound - blocks_e) * block_rows
    rank_f = jnp.sum(prefix_f * ohf, axis=1) - 1.0
    pos = (jnp.sum(pad_start.astype(jnp.float32)[None, :] * ohf, axis=1)
           + rank_f).astype(jnp.int32)
    p_total = s + e_num * block_rows
    nb = p_total // block_rows
    block_expert = jnp.minimum(
        jnp.sum((block_bound[None, :] <=
                 jnp.arange(nb, dtype=jnp.int32)[:, None]).astype(jnp.int32),
                axis=1),
        e_num - 1).astype(jnp.int32)
    num_used = block_bound[-1].astype(jnp.int32)
    return pos, block_expert, num_used


def _sc_dispatch_rows(table, pos, p_total):
    m, k = table.shape
    topk = pos.shape[0] // m
    info = plsc.get_sparse_core_info()
    nw = info.num_cores * info.num_subcores
    toks_w = m // nw
    ct = 32
    chunks = toks_w // ct
    pos4 = pos.reshape(nw, chunks, ct, topk).transpose(0, 1, 3, 2)
    mesh = plsc.VectorSubcoreMesh(core_axis_name="c", subcore_axis_name="s")

    @functools.partial(
        pl.kernel, mesh=mesh,
        out_type=jax.ShapeDtypeStruct((p_total, k), table.dtype),
        scratch_types=[pltpu.VMEM((chunks, topk, ct), jnp.int32)]
                      + [pltpu.VMEM((ct, k), table.dtype) for _ in range(2)]
                      + [pltpu.SemaphoreType.DMA for _ in range(2 + 2 * topk)])
    def kern(table_hbm, idx_hbm, out_hbm, idx_v, *rest):
        bufs = rest[:2]
        lsems = rest[2:4]
        ssems = (rest[4:4 + topk], rest[4 + topk:4 + 2 * topk])
        wid = jax.lax.axis_index("s") * info.num_cores + jax.lax.axis_index("c")
        base = wid * toks_w
        pltpu.sync_copy(idx_hbm.at[wid], idx_v)
        lh = [None] * chunks
        sh = [None] * chunks
        for c in range(chunks):
            if c >= 2:
                for h in sh[c - 2]:
                    h.wait()
            lh[c] = pltpu.async_copy(
                table_hbm.at[pl.ds(base + c * ct, ct)], bufs[c % 2],
                lsems[c % 2])
            if c >= 1:
                lh[c - 1].wait()
                sh[c - 1] = [
                    pltpu.async_copy(bufs[(c - 1) % 2],
                                     out_hbm.at[idx_v.at[c - 1, t]],
                                     ssems[(c - 1) % 2][t])
                    for t in range(topk)]
        lh[chunks - 1].wait()
        sh[chunks - 1] = [
            pltpu.async_copy(bufs[(chunks - 1) % 2],
                             out_hbm.at[idx_v.at[chunks - 1, t]],
                             ssems[(chunks - 1) % 2][t])
            for t in range(topk)]
        for c in (chunks - 2, chunks - 1):
            for h in sh[c]:
                h.wait()

    return kern(table, pos4)


def _sc_combine_rows(yw, pos, topk_weights, m, topk):
    k = yw.shape[1]
    info = plsc.get_sparse_core_info()
    nw = info.num_cores * info.num_subcores
    nl = info.num_lanes
    toks_w = m // nw
    cc = 16
    chunks = toks_w // cc
    pos4 = pos.reshape(nw, chunks, cc, topk).transpose(0, 1, 3, 2)
    wbc = jnp.broadcast_to(topk_weights.reshape(m, topk, 1), (m, topk, nl))
    wbc = wbc.reshape(nw, chunks, cc, topk, nl).transpose(0, 1, 3, 2, 4)
    mesh = plsc.VectorSubcoreMesh(core_axis_name="c", subcore_axis_name="s")

    @functools.partial(
        pl.kernel, mesh=mesh,
        out_type=jax.ShapeDtypeStruct((m, k), yw.dtype),
        scratch_types=[pltpu.VMEM((chunks, topk, cc), jnp.int32),
                       pltpu.VMEM((chunks, topk, cc, nl), jnp.float32)]
                      + [pltpu.VMEM((cc, k), jnp.float32)
                         for _ in range(2 * topk)]
                      + [pltpu.SemaphoreType.DMA for _ in range(2 * topk + 2)])
    def kern(y_hbm, p_hbm, w_hbm, o_hbm, idx_v, w_v, *rest):
        bufs = (rest[:topk], rest[topk:2 * topk])
        gsems = (rest[2 * topk:3 * topk], rest[3 * topk:4 * topk])
        wsems = rest[4 * topk:4 * topk + 2]
        wid = jax.lax.axis_index("s") * info.num_cores + jax.lax.axis_index("c")
        base = wid * toks_w
        pltpu.sync_copy(p_hbm.at[wid], idx_v)
        pltpu.sync_copy(w_hbm.at[wid], w_v)
        gh = [None] * chunks
        wh = [None] * chunks

        def accumulate(c):
            bb = bufs[c % 2]

            @pl.loop(0, cc)
            def _(r):
                @pl.loop(0, k, step=nl)
                def _(col):
                    slc = (pl.ds(r, 1), pl.ds(col, nl))
                    acc = (bb[0].at[*slc][...]
                           * w_v.at[c, 0, pl.ds(r, 1)][...])
                    for t in range(1, topk):
                        acc = acc + (bb[t].at[*slc][...]
                                     * w_v.at[c, t, pl.ds(r, 1)][...])
                    bb[0].at[*slc][...] = acc

        for c in range(chunks):
            if c >= 2:
                wh[c - 2].wait()
            gh[c] = [pltpu.async_copy(y_hbm.at[idx_v.at[c, t]],
                                      bufs[c % 2][t], gsems[c % 2][t])
                     for t in range(topk)]
            if c >= 1:
                for h in gh[c - 1]:
                    h.wait()
                accumulate(c - 1)
                wh[c - 1] = pltpu.async_copy(
                    bufs[(c - 1) % 2][0],
                    o_hbm.at[pl.ds(base + (c - 1) * cc, cc)], wsems[(c - 1) % 2])
        for h in gh[chunks - 1]:
            h.wait()
        accumulate(chunks - 1)
        wh[chunks - 1] = pltpu.async_copy(
            bufs[(chunks - 1) % 2][0],
            o_hbm.at[pl.ds(base + (chunks - 1) * cc, cc)], wsems[(chunks - 1) % 2])
        wh[chunks - 2].wait()
        wh[chunks - 1].wait()

    return kern(yw, pos4, wbc)


def _expert_gemm(xw, w1, w2, block_expert, num_used):
    p_total, k = xw.shape
    e_num, n, _ = w1.shape
    d_ff = n // 2
    t_steps = d_ff // _TN
    nb = p_total // _B

    def body(be_ref, nu_ref, x_ref, w1g_ref, w1u_ref, w2_ref, o_ref,
             acc_ref):
        b = pl.program_id(0)
        t = pl.program_id(1)

        @pl.when(b < nu_ref[0])
        def _():
            x = x_ref[...]
            g = jax.lax.dot_general(x, w1g_ref[0], (((1,), (1,)), ((), ())),
                                    preferred_element_type=jnp.float32)
            u = jax.lax.dot_general(x, w1u_ref[0], (((1,), (1,)), ((), ())),
                                    preferred_element_type=jnp.float32)
            act = g * jax.nn.sigmoid(g) * u
            y = jax.lax.dot_general(act, w2_ref[0], (((1,), (1,)), ((), ())),
                                    preferred_element_type=jnp.float32)

            @pl.when(t == 0)
            def _():
                acc_ref[...] = y

            @pl.when(t != 0)
            def _():
                acc_ref[...] += y

            @pl.when(t == t_steps - 1)
            def _():
                o_ref[...] = acc_ref[...]

    def beff(b, nu):
        return jnp.minimum(b, nu[0] - 1)

    def eeff(b, be, nu):
        return be[jnp.minimum(b, nu[0] - 1)]

    def teff(b, t, nu):
        return jnp.where(b < nu[0], t, t_steps - 1)

    grid_spec = pltpu.PrefetchScalarGridSpec(
        num_scalar_prefetch=2,
        grid=(nb, t_steps),
        in_specs=[
            pl.BlockSpec((_B, k), lambda b, t, be, nu: (beff(b, nu), 0)),
            pl.BlockSpec((1, _TN, k),
                         lambda b, t, be, nu: (eeff(b, be, nu), teff(b, t, nu), 0)),
            pl.BlockSpec((1, _TN, k),
                         lambda b, t, be, nu: (eeff(b, be, nu),
                                               t_steps + teff(b, t, nu), 0)),
            pl.BlockSpec((1, k, _TN),
                         lambda b, t, be, nu: (eeff(b, be, nu), 0, teff(b, t, nu))),
        ],
        out_specs=pl.BlockSpec((_B, k), lambda b, t, be, nu: (b, 0)),
        scratch_shapes=[pltpu.VMEM((_B, k), jnp.float32)],
    )
    return pl.pallas_call(
        body,
        grid_spec=grid_spec,
        out_shape=jax.ShapeDtypeStruct((p_total, k), jnp.float32),
        compiler_params=pltpu.CompilerParams(
            dimension_semantics=("parallel", "arbitrary")),
    )(block_expert, num_used.reshape(1), xw, w1, w1, w2)


def kernel(hidden_states, w1, w2, topk_weights, topk_ids):
    m, _ = hidden_states.shape
    e_num = w1.shape[0]
    topk = topk_ids.shape[1]
    s = m * topk
    pos, block_expert, num_used = _routing_metadata(
        topk_ids, topk_weights, e_num, _B)
    p_total = s + e_num * _B
    xw = _sc_dispatch_rows(hidden_states, pos, p_total)
    yw = _expert_gemm(xw, w1, w2, block_expert, num_used)
    return _sc_combine_rows(yw, pos, topk_weights, m, topk)


# speedup vs baseline: 1.0289x; 1.0229x over previous
import functools

import jax
import jax.numpy as jnp
from jax.experimental import pallas as pl
from jax.experimental.pallas import tpu as pltpu
from jax.experimental.pallas import tpu_sc as plsc

_B = 512
_TN = 1024


def _routing_metadata(topk_ids, topk_weights, e_num, block_rows):
    m, topk = topk_ids.shape
    s = m * topk
    flat_e = topk_ids.reshape(s).astype(jnp.int32)
    chunk = 128
    g = s // chunk
    ohf = (flat_e[:, None] == jnp.arange(e_num, dtype=jnp.int32)[None, :]
           ).astype(jnp.float32)
    ohc = ohf.reshape(g, chunk, e_num)
    tri = jnp.tril(jnp.ones((chunk, chunk), jnp.float32))
    inner = jax.lax.dot_general(tri, ohc, (((1,), (1,)), ((), ())))
    inner = inner.transpose(1, 0, 2)
    chunk_tot = ohc.sum(axis=1)
    outer = jnp.cumsum(chunk_tot, axis=0) - chunk_tot
    prefix_f = (inner + outer[:, None, :]).reshape(s, e_num)
    counts = prefix_f[-1].astype(jnp.int32)
    blocks_e = (counts + block_rows - 1) // block_rows
    block_bound = jnp.cumsum(blocks_e)
    pad_start = (block_bound - blocks_e) * block_rows
    rank_f = jnp.sum(prefix_f * ohf, axis=1) - 1.0
    pos = (jnp.sum(pad_start.astype(jnp.float32)[None, :] * ohf, axis=1)
           + rank_f).astype(jnp.int32)
    p_total = s + e_num * block_rows
    nb = p_total // block_rows
    block_expert = jnp.minimum(
        jnp.sum((block_bound[None, :] <=
                 jnp.arange(nb, dtype=jnp.int32)[:, None]).astype(jnp.int32),
                axis=1),
        e_num - 1).astype(jnp.int32)
    num_used = block_bound[-1].astype(jnp.int32)
    return pos, block_expert, num_used


def _sc_dispatch_rows(table, pos, p_total):
    m, k = table.shape
    topk = pos.shape[0] // m
    info = plsc.get_sparse_core_info()
    nw = info.num_cores * info.num_subcores
    toks_w = m // nw
    ct = 32
    chunks = toks_w // ct
    pos4 = pos.reshape(nw, chunks, ct, topk).transpose(0, 1, 3, 2)
    mesh = plsc.VectorSubcoreMesh(core_axis_name="c", subcore_axis_name="s")

    @functools.partial(
        pl.kernel, mesh=mesh,
        out_type=jax.ShapeDtypeStruct((p_total, k), table.dtype),
        scratch_types=[pltpu.VMEM((chunks, topk, ct), jnp.int32)]
                      + [pltpu.VMEM((ct, k), table.dtype) for _ in range(2)]
                      + [pltpu.SemaphoreType.DMA for _ in range(2 + 2 * topk)])
    def kern(table_hbm, idx_hbm, out_hbm, idx_v, *rest):
        bufs = rest[:2]
        lsems = rest[2:4]
        ssems = (rest[4:4 + topk], rest[4 + topk:4 + 2 * topk])
        wid = jax.lax.axis_index("s") * info.num_cores + jax.lax.axis_index("c")
        base = wid * toks_w
        pltpu.sync_copy(idx_hbm.at[wid], idx_v)
        lh = [None] * chunks
        sh = [None] * chunks
        for c in range(chunks):
            if c >= 2:
                for h in sh[c - 2]:
                    h.wait()
            lh[c] = pltpu.async_copy(
                table_hbm.at[pl.ds(base + c * ct, ct)], bufs[c % 2],
                lsems[c % 2])
            if c >= 1:
                lh[c - 1].wait()
                sh[c - 1] = [
                    pltpu.async_copy(bufs[(c - 1) % 2],
                                     out_hbm.at[idx_v.at[c - 1, t]],
                                     ssems[(c - 1) % 2][t])
                    for t in range(topk)]
        lh[chunks - 1].wait()
        sh[chunks - 1] = [
            pltpu.async_copy(bufs[(chunks - 1) % 2],
                             out_hbm.at[idx_v.at[chunks - 1, t]],
                             ssems[(chunks - 1) % 2][t])
            for t in range(topk)]
        for c in (chunks - 2, chunks - 1):
            for h in sh[c]:
                h.wait()

    return kern(table, pos4)


def _sc_combine_rows(yw, pos, topk_weights, m, topk):
    k = yw.shape[1]
    info = plsc.get_sparse_core_info()
    nw = info.num_cores * info.num_subcores
    nl = info.num_lanes
    toks_w = m // nw
    cc = 16
    chunks = toks_w // cc
    pos4 = pos.reshape(nw, chunks, cc, topk).transpose(0, 1, 3, 2)
    wbc = jnp.broadcast_to(topk_weights.reshape(m, topk, 1), (m, topk, nl))
    wbc = wbc.reshape(nw, chunks, cc, topk, nl).transpose(0, 1, 3, 2, 4)
    mesh = plsc.VectorSubcoreMesh(core_axis_name="c", subcore_axis_name="s")

    @functools.partial(
        pl.kernel, mesh=mesh,
        out_type=jax.ShapeDtypeStruct((m, k), yw.dtype),
        scratch_types=[pltpu.VMEM((chunks, topk, cc), jnp.int32),
                       pltpu.VMEM((chunks, topk, cc, nl), jnp.float32)]
                      + [pltpu.VMEM((cc, k), jnp.float32)
                         for _ in range(2 * topk)]
                      + [pltpu.SemaphoreType.DMA for _ in range(2 * topk + 2)])
    def kern(y_hbm, p_hbm, w_hbm, o_hbm, idx_v, w_v, *rest):
        bufs = (rest[:topk], rest[topk:2 * topk])
        gsems = (rest[2 * topk:3 * topk], rest[3 * topk:4 * topk])
        wsems = rest[4 * topk:4 * topk + 2]
        wid = jax.lax.axis_index("s") * info.num_cores + jax.lax.axis_index("c")
        base = wid * toks_w
        pltpu.sync_copy(p_hbm.at[wid], idx_v)
        pltpu.sync_copy(w_hbm.at[wid], w_v)
        gh = [None] * chunks
        wh = [None] * chunks

        def accumulate(c):
            bb = bufs[c % 2]

            @pl.loop(0, cc)
            def _(r):
                wts = [w_v.at[c, t, pl.ds(r, 1)][...] for t in range(topk)]

                @pl.loop(0, k, step=nl)
                def _(col):
                    slc = (pl.ds(r, 1), pl.ds(col, nl))
                    acc = bb[0].at[*slc][...] * wts[0]
                    for t in range(1, topk):
                        acc = acc + bb[t].at[*slc][...] * wts[t]
                    bb[0].at[*slc][...] = acc

        for c in range(chunks):
            if c >= 2:
                wh[c - 2].wait()
            gh[c] = [pltpu.async_copy(y_hbm.at[idx_v.at[c, t]],
                                      bufs[c % 2][t], gsems[c % 2][t])
                     for t in range(topk)]
            if c >= 1:
                for h in gh[c - 1]:
                    h.wait()
                accumulate(c - 1)
                wh[c - 1] = pltpu.async_copy(
                    bufs[(c - 1) % 2][0],
                    o_hbm.at[pl.ds(base + (c - 1) * cc, cc)], wsems[(c - 1) % 2])
        for h in gh[chunks - 1]:
            h.wait()
        accumulate(chunks - 1)
        wh[chunks - 1] = pltpu.async_copy(
            bufs[(chunks - 1) % 2][0],
            o_hbm.at[pl.ds(base + (chunks - 1) * cc, cc)], wsems[(chunks - 1) % 2])
        wh[chunks - 2].wait()
        wh[chunks - 1].wait()

    return kern(yw, pos4, wbc)


def _expert_gemm(xw, w1, w2, block_expert, num_used):
    p_total, k = xw.shape
    e_num, n, _ = w1.shape
    d_ff = n // 2
    t_steps = d_ff // _TN
    nb = p_total // _B

    def body(be_ref, nu_ref, x_ref, w1g_ref, w1u_ref, w2_ref, o_ref,
             acc_ref):
        b = pl.program_id(0)
        t = pl.program_id(1)

        @pl.when(b < nu_ref[0])
        def _():
            x = x_ref[...]
            g = jax.lax.dot_general(x, w1g_ref[0], (((1,), (1,)), ((), ())),
                                    preferred_element_type=jnp.float32)
            u = jax.lax.dot_general(x, w1u_ref[0], (((1,), (1,)), ((), ())),
                                    preferred_element_type=jnp.float32)
            act = g * jax.nn.sigmoid(g) * u
            y = jax.lax.dot_general(act, w2_ref[0], (((1,), (1,)), ((), ())),
                                    preferred_element_type=jnp.float32)

            @pl.when(t == 0)
            def _():
                acc_ref[...] = y

            @pl.when(t != 0)
            def _():
                acc_ref[...] += y

            @pl.when(t == t_steps - 1)
            def _():
                o_ref[...] = acc_ref[...]

    def beff(b, nu):
        return jnp.minimum(b, nu[0] - 1)

    def eeff(b, be, nu):
        return be[jnp.minimum(b, nu[0] - 1)]

    def teff(b, t, nu):
        return jnp.where(b < nu[0], t, t_steps - 1)

    grid_spec = pltpu.PrefetchScalarGridSpec(
        num_scalar_prefetch=2,
        grid=(nb, t_steps),
        in_specs=[
            pl.BlockSpec((_B, k), lambda b, t, be, nu: (beff(b, nu), 0)),
            pl.BlockSpec((1, _TN, k),
                         lambda b, t, be, nu: (eeff(b, be, nu), teff(b, t, nu), 0)),
            pl.BlockSpec((1, _TN, k),
                         lambda b, t, be, nu: (eeff(b, be, nu),
                                               t_steps + teff(b, t, nu), 0)),
            pl.BlockSpec((1, k, _TN),
                         lambda b, t, be, nu: (eeff(b, be, nu), 0, teff(b, t, nu))),
        ],
        out_specs=pl.BlockSpec((_B, k), lambda b, t, be, nu: (b, 0)),
        scratch_shapes=[pltpu.VMEM((_B, k), jnp.float32)],
    )
    return pl.pallas_call(
        body,
        grid_spec=grid_spec,
        out_shape=jax.ShapeDtypeStruct((p_total, k), jnp.float32),
        compiler_params=pltpu.CompilerParams(
            dimension_semantics=("parallel", "arbitrary")),
    )(block_expert, num_used.reshape(1), xw, w1, w1, w2)


def kernel(hidden_states, w1, w2, topk_weights, topk_ids):
    m, _ = hidden_states.shape
    e_num = w1.shape[0]
    topk = topk_ids.shape[1]
    s = m * topk
    pos, block_expert, num_used = _routing_metadata(
        topk_ids, topk_weights, e_num, _B)
    p_total = s + e_num * _B
    xw = _sc_dispatch_rows(hidden_states, pos, p_total)
    yw = _expert_gemm(xw, w1, w2, block_expert, num_used)
    return _sc_combine_rows(yw, pos, topk_weights, m, topk)


# speedup vs baseline: 1.1232x; 1.0916x over previous
import functools

import jax
import jax.numpy as jnp
from jax.experimental import pallas as pl
from jax.experimental.pallas import tpu as pltpu
from jax.experimental.pallas import tpu_sc as plsc

_B = 512
_TN = 1024


def _routing_metadata(topk_ids, topk_weights, e_num, block_rows):
    m, topk = topk_ids.shape
    s = m * topk
    flat_e = topk_ids.reshape(s).astype(jnp.int32)
    chunk = 128
    g = s // chunk
    ohf = (flat_e[:, None] == jnp.arange(e_num, dtype=jnp.int32)[None, :]
           ).astype(jnp.float32)
    ohc = ohf.reshape(g, chunk, e_num)
    tri = jnp.tril(jnp.ones((chunk, chunk), jnp.float32))
    inner = jax.lax.dot_general(tri, ohc, (((1,), (1,)), ((), ())))
    inner = inner.transpose(1, 0, 2)
    chunk_tot = ohc.sum(axis=1)
    outer = jnp.cumsum(chunk_tot, axis=0) - chunk_tot
    prefix_f = (inner + outer[:, None, :]).reshape(s, e_num)
    counts = prefix_f[-1].astype(jnp.int32)
    blocks_e = (counts + block_rows - 1) // block_rows
    block_bound = jnp.cumsum(blocks_e)
    pad_start = (block_bound - blocks_e) * block_rows
    rank_f = jnp.sum(prefix_f * ohf, axis=1) - 1.0
    pos = (jnp.sum(pad_start.astype(jnp.float32)[None, :] * ohf, axis=1)
           + rank_f).astype(jnp.int32)
    p_total = s + e_num * block_rows
    nb = p_total // block_rows
    block_expert = jnp.minimum(
        jnp.sum((block_bound[None, :] <=
                 jnp.arange(nb, dtype=jnp.int32)[:, None]).astype(jnp.int32),
                axis=1),
        e_num - 1).astype(jnp.int32)
    num_used = block_bound[-1].astype(jnp.int32)
    return pos, block_expert, num_used


def _sc_dispatch_rows(table, pos, p_total):
    m, k = table.shape
    topk = pos.shape[0] // m
    info = plsc.get_sparse_core_info()
    nw = info.num_cores * info.num_subcores
    toks_w = m // nw
    ct = 32
    chunks = toks_w // ct
    pos4 = pos.reshape(nw, chunks, ct, topk).transpose(0, 1, 3, 2)
    mesh = plsc.VectorSubcoreMesh(core_axis_name="c", subcore_axis_name="s")

    @functools.partial(
        pl.kernel, mesh=mesh,
        out_type=jax.ShapeDtypeStruct((p_total, k), table.dtype),
        scratch_types=[pltpu.VMEM((chunks, topk, ct), jnp.int32)]
                      + [pltpu.VMEM((ct, k), table.dtype) for _ in range(2)]
                      + [pltpu.SemaphoreType.DMA for _ in range(2 + 2 * topk)])
    def kern(table_hbm, idx_hbm, out_hbm, idx_v, *rest):
        bufs = rest[:2]
        lsems = rest[2:4]
        ssems = (rest[4:4 + topk], rest[4 + topk:4 + 2 * topk])
        wid = jax.lax.axis_index("s") * info.num_cores + jax.lax.axis_index("c")
        base = wid * toks_w
        pltpu.sync_copy(idx_hbm.at[wid], idx_v)
        lh = [None] * chunks
        sh = [None] * chunks
        for c in range(chunks):
            if c >= 2:
                for h in sh[c - 2]:
                    h.wait()
            lh[c] = pltpu.async_copy(
                table_hbm.at[pl.ds(base + c * ct, ct)], bufs[c % 2],
                lsems[c % 2])
            if c >= 1:
                lh[c - 1].wait()
                sh[c - 1] = [
                    pltpu.async_copy(bufs[(c - 1) % 2],
                                     out_hbm.at[idx_v.at[c - 1, t]],
                                     ssems[(c - 1) % 2][t])
                    for t in range(topk)]
        lh[chunks - 1].wait()
        sh[chunks - 1] = [
            pltpu.async_copy(bufs[(chunks - 1) % 2],
                             out_hbm.at[idx_v.at[chunks - 1, t]],
                             ssems[(chunks - 1) % 2][t])
            for t in range(topk)]
        for c in (chunks - 2, chunks - 1):
            for h in sh[c]:
                h.wait()

    return kern(table, pos4)


def _sc_combine_rows(yw, pos, topk_weights, m, topk):
    k = yw.shape[1]
    info = plsc.get_sparse_core_info()
    nw = info.num_cores * info.num_subcores
    nl = info.num_lanes
    toks_w = m // nw
    cc = 16
    chunks = toks_w // cc
    pos4 = pos.reshape(nw, chunks, cc, topk).transpose(0, 1, 3, 2)
    wbc = jnp.broadcast_to(topk_weights.reshape(m, topk, 1), (m, topk, nl))
    wbc = wbc.reshape(nw, chunks, cc, topk, nl).transpose(0, 1, 3, 2, 4)
    mesh = plsc.VectorSubcoreMesh(core_axis_name="c", subcore_axis_name="s")

    @functools.partial(
        pl.kernel, mesh=mesh,
        out_type=jax.ShapeDtypeStruct((m, k), yw.dtype),
        scratch_types=[pltpu.VMEM((chunks, topk, cc), jnp.int32),
                       pltpu.VMEM((chunks, topk, cc, nl), jnp.float32)]
                      + [pltpu.VMEM((cc, k), jnp.float32)
                         for _ in range(2 * topk)]
                      + [pltpu.SemaphoreType.DMA for _ in range(2 * topk + 2)])
    def kern(y_hbm, p_hbm, w_hbm, o_hbm, idx_v, w_v, *rest):
        bufs = (rest[:topk], rest[topk:2 * topk])
        gsems = (rest[2 * topk:3 * topk], rest[3 * topk:4 * topk])
        wsems = rest[4 * topk:4 * topk + 2]
        wid = jax.lax.axis_index("s") * info.num_cores + jax.lax.axis_index("c")
        base = wid * toks_w
        pltpu.sync_copy(p_hbm.at[wid], idx_v)
        pltpu.sync_copy(w_hbm.at[wid], w_v)
        gh = [None] * chunks
        wh = [None] * chunks

        def accumulate(c):
            bb = bufs[c % 2]

            @pl.loop(0, cc)
            def _(r):
                wts = [w_v.at[c, t, pl.ds(r, 1)][...] for t in range(topk)]

                @pl.loop(0, k, step=4 * nl)
                def _(col):
                    for sub in range(4):
                        slc = (pl.ds(r, 1), pl.ds(col + sub * nl, nl))
                        acc = bb[0].at[*slc][...] * wts[0]
                        for t in range(1, topk):
                            acc = acc + bb[t].at[*slc][...] * wts[t]
                        bb[0].at[*slc][...] = acc

        for c in range(chunks):
            if c >= 2:
                wh[c - 2].wait()
            gh[c] = [pltpu.async_copy(y_hbm.at[idx_v.at[c, t]],
                                      bufs[c % 2][t], gsems[c % 2][t])
                     for t in range(topk)]
            if c >= 1:
                for h in gh[c - 1]:
                    h.wait()
                accumulate(c - 1)
                wh[c - 1] = pltpu.async_copy(
                    bufs[(c - 1) % 2][0],
                    o_hbm.at[pl.ds(base + (c - 1) * cc, cc)], wsems[(c - 1) % 2])
        for h in gh[chunks - 1]:
            h.wait()
        accumulate(chunks - 1)
        wh[chunks - 1] = pltpu.async_copy(
            bufs[(chunks - 1) % 2][0],
            o_hbm.at[pl.ds(base + (chunks - 1) * cc, cc)], wsems[(chunks - 1) % 2])
        wh[chunks - 2].wait()
        wh[chunks - 1].wait()

    return kern(yw, pos4, wbc)


def _expert_gemm(xw, w1, w2, block_expert, num_used):
    p_total, k = xw.shape
    e_num, n, _ = w1.shape
    d_ff = n // 2
    t_steps = d_ff // _TN
    nb = p_total // _B

    def body(be_ref, nu_ref, x_ref, w1g_ref, w1u_ref, w2_ref, o_ref,
             acc_ref):
        b = pl.program_id(0)
        t = pl.program_id(1)

        @pl.when(b < nu_ref[0])
        def _():
            x = x_ref[...]
            g = jax.lax.dot_general(x, w1g_ref[0], (((1,), (1,)), ((), ())),
                                    preferred_element_type=jnp.float32)
            u = jax.lax.dot_general(x, w1u_ref[0], (((1,), (1,)), ((), ())),
                                    preferred_element_type=jnp.float32)
            act = g * jax.nn.sigmoid(g) * u
            y = jax.lax.dot_general(act, w2_ref[0], (((1,), (1,)), ((), ())),
                                    preferred_element_type=jnp.float32)

            @pl.when(t == 0)
            def _():
                acc_ref[...] = y

            @pl.when(t != 0)
            def _():
                acc_ref[...] += y

            @pl.when(t == t_steps - 1)
            def _():
                o_ref[...] = acc_ref[...]

    def beff(b, nu):
        return jnp.minimum(b, nu[0] - 1)

    def eeff(b, be, nu):
        return be[jnp.minimum(b, nu[0] - 1)]

    def teff(b, t, nu):
        return jnp.where(b < nu[0], t, t_steps - 1)

    grid_spec = pltpu.PrefetchScalarGridSpec(
        num_scalar_prefetch=2,
        grid=(nb, t_steps),
        in_specs=[
            pl.BlockSpec((_B, k), lambda b, t, be, nu: (beff(b, nu), 0)),
            pl.BlockSpec((1, _TN, k),
                         lambda b, t, be, nu: (eeff(b, be, nu), teff(b, t, nu), 0)),
            pl.BlockSpec((1, _TN, k),
                         lambda b, t, be, nu: (eeff(b, be, nu),
                                               t_steps + teff(b, t, nu), 0)),
            pl.BlockSpec((1, k, _TN),
                         lambda b, t, be, nu: (eeff(b, be, nu), 0, teff(b, t, nu))),
        ],
        out_specs=pl.BlockSpec((_B, k), lambda b, t, be, nu: (b, 0)),
        scratch_shapes=[pltpu.VMEM((_B, k), jnp.float32)],
    )
    return pl.pallas_call(
        body,
        grid_spec=grid_spec,
        out_shape=jax.ShapeDtypeStruct((p_total, k), jnp.float32),
        compiler_params=pltpu.CompilerParams(
            dimension_semantics=("parallel", "arbitrary")),
    )(block_expert, num_used.reshape(1), xw, w1, w1, w2)


def kernel(hidden_states, w1, w2, topk_weights, topk_ids):
    m, _ = hidden_states.shape
    e_num = w1.shape[0]
    topk = topk_ids.shape[1]
    s = m * topk
    pos, block_expert, num_used = _routing_metadata(
        topk_ids, topk_weights, e_num, _B)
    p_total = s + e_num * _B
    xw = _sc_dispatch_rows(hidden_states, pos, p_total)
    yw = _expert_gemm(xw, w1, w2, block_expert, num_used)
    return _sc_combine_rows(yw, pos, topk_weights, m, topk)
